# baseline (speedup 1.0000x reference)
import functools

import jax
import jax.numpy as jnp
import numpy as np
from jax import lax
from jax.experimental import pallas as pl
from jax.experimental.pallas import tpu as pltpu
from jax.experimental.pallas import tpu_sc as plsc

N_NODES = 10000
N_GRAPHS = 256
DIM = 128
NTOT = N_NODES + N_GRAPHS
E_ORIG = 320000
E_EXT = E_ORIG + 2 * N_NODES

NC, NS = 2, 16
NW = NC * NS
KC = 128
CHUNKS = -(-E_EXT // (NW * KC))
IDXBLK = 12
NBLK = CHUNKS // IDXBLK
E_PAD = NW * KC * CHUNKS
ZROWS = 16
ZCOPIES = 41
TILE_ACC_ROWS = ZROWS * ZCOPIES
ACC_ROWS = TILE_ACC_ROWS * NS
DUMP_ROWS = TILE_ACC_ROWS
PAD_ROWS = ACC_ROWS
NB = 8
BR = PAD_ROWS // NB


def _sc_aggr_kernel(width):
  mesh = plsc.VectorSubcoreMesh(core_axis_name="c", subcore_axis_name="s")

  def body(table_hbm, src_hbm, dst_hbm, out_hbm,
           src_v, dst_v, rows_a, rows_b, acc_sh, zbuf,
           gsem_a, gsem_b, ssem_a, ssem_b, isem, zsem):
    c = lax.axis_index("c")
    s = lax.axis_index("s")
    w = c * NS + s
    zv = jnp.zeros((16,), jnp.float32)
    for i in range(ZROWS):
      for t in range(width // 16):
        zbuf[i, pl.ds(16 * t, 16)] = zv
    zdescs = []
    for r in range(ZCOPIES):
      zdescs.append(pltpu.async_copy(
          zbuf, acc_sh.at[pl.ds(s * TILE_ACC_ROWS + r * ZROWS, ZROWS)],
          zsem))
    pltpu.sync_copy(src_hbm.at[w].at[0], src_v.at[0])
    pltpu.sync_copy(dst_hbm.at[w].at[0], dst_v.at[0])
    idescs = {}

    def src_row(j):
      return src_v.at[(j // IDXBLK) % 2].at[j % IDXBLK]

    def dst_row(j):
      return dst_v.at[(j // IDXBLK) % 2].at[j % IDXBLK]

    for d in zdescs:
      d.wait()
    plsc.subcore_barrier()
    bufs = (rows_a, rows_b)
    gsems = (gsem_a, gsem_b)
    ssems = (ssem_a, ssem_b)
    gdesc = [None, None]
    sdesc = [None, None]
    gdesc[0] = pltpu.async_copy(table_hbm.at[src_row(0)], bufs[0], gsems[0])
    for j in range(CHUNKS):
      i = j % 2
      b, k = divmod(j, IDXBLK)
      if k == 2 and b + 1 < NBLK:
        nb = b + 1
        idescs[nb] = (
            pltpu.async_copy(src_hbm.at[w].at[nb], src_v.at[nb % 2], isem),
            pltpu.async_copy(dst_hbm.at[w].at[nb], dst_v.at[nb % 2], isem))
      gdesc[i].wait()
      sdesc[i] = pltpu.async_copy(bufs[i], acc_sh.at[dst_row(j)],
                                  ssems[i], add=True)
      if j + 1 < CHUNKS:
        if sdesc[1 - i] is not None:
          sdesc[1 - i].wait()
        if (j + 1) % IDXBLK == 0:
          for d in idescs.pop((j + 1) // IDXBLK):
            d.wait()
        gdesc[1 - i] = pltpu.async_copy(table_hbm.at[src_row(j + 1)],
                                        bufs[1 - i], gsems[1 - i])
    sdesc[(CHUNKS - 1) % 2].wait()
    if sdesc[CHUNKS % 2] is not None:
      sdesc[CHUNKS % 2].wait()
    plsc.subcore_barrier()
    pltpu.sync_copy(acc_sh.at[pl.ds(s * DUMP_ROWS, DUMP_ROWS)],
                    out_hbm.at[c].at[pl.ds(s * DUMP_ROWS, DUMP_ROWS)])

  return pl.kernel(
      body,
      out_type=jax.ShapeDtypeStruct((NC, ACC_ROWS, width), jnp.float32),
      mesh=mesh,
      scratch_types=[
          pltpu.VMEM((2, IDXBLK, KC), jnp.int32),
          pltpu.VMEM((2, IDXBLK, KC), jnp.int32),
          pltpu.VMEM((KC, width), jnp.float32),
          pltpu.VMEM((KC, width), jnp.float32),
          pltpu.VMEM_SHARED((ACC_ROWS, width), jnp.float32),
          pltpu.VMEM((ZROWS, width), jnp.float32),
          pltpu.SemaphoreType.DMA,
          pltpu.SemaphoreType.DMA,
          pltpu.SemaphoreType.DMA,
          pltpu.SemaphoreType.DMA,
          pltpu.SemaphoreType.DMA,
          pltpu.SemaphoreType.DMA,
      ],
  )


def _embed_body(xi_ref, emb1_ref, emb2_ref, teb_ref, out_ref):
  xi0 = xi_ref[:, 0:1]
  xi1 = xi_ref[:, 1:2]
  oh0 = (xi0 == lax.broadcasted_iota(jnp.int32, (1, 120), 1)).astype(jnp.float32)
  oh1 = (xi1 == lax.broadcasted_iota(jnp.int32, (1, 8), 1)).astype(jnp.float32)
  x0 = jnp.dot(oh0, emb1_ref[...], preferred_element_type=jnp.float32, precision=lax.Precision.HIGHEST)
  x0 = x0 + jnp.dot(oh1, emb2_ref[...], preferred_element_type=jnp.float32, precision=lax.Precision.HIGHEST)
  out_ref[pl.ds(0, N_NODES), :] = x0
  out_ref[pl.ds(N_NODES, N_GRAPHS), :] = teb_ref[...]
  out_ref[pl.ds(NTOT, PAD_ROWS - NTOT), :] = jnp.zeros(
      (PAD_ROWS - NTOT, DIM), jnp.float32)


def _mlp_body(film, p0_ref, p1_ref, xc_ref, cc_ref,
              a16_ref, b16_ref, e1_ref, e2_ref, w1_ref, b1_ref,
              w2_ref, b2_ref, te_ref, wg_ref, bg_ref,
              wb_ref, bb_ref, bp_ref, z_ref, st_ref):
  i = pl.program_id(0)
  hp = lax.Precision.HIGHEST
  e1 = e1_ref[...]
  e2 = e2_ref[...]
  t = (jnp.dot(a16_ref[...], e1, preferred_element_type=jnp.float32, precision=hp) +
       jnp.dot(b16_ref[...], e2, preferred_element_type=jnp.float32, precision=hp))
  const = e1[4:5, :] + e2[0:1, :]
  aggr = (p0_ref[0] + p1_ref[0] + xc_ref[...] + const +
          jnp.dot(cc_ref[...], t, preferred_element_type=jnp.float32, precision=hp))
  h = jnp.maximum(
      jnp.dot(aggr, w1_ref[...], preferred_element_type=jnp.float32, precision=hp) +
      b1_ref[...], 0.0)
  y = jnp.dot(h, w2_ref[...], preferred_element_type=jnp.float32, precision=hp) + b2_ref[...]
  grow = i * BR + lax.broadcasted_iota(jnp.int32, (BR, 1), 0)
  if film:
    gam = jnp.dot(te_ref[...], wg_ref[...],
                  preferred_element_type=jnp.float32, precision=hp) + bg_ref[...]
    bet = jnp.dot(te_ref[...], wb_ref[...],
                  preferred_element_type=jnp.float32, precision=hp) + bb_ref[...]
    oh = jnp.logical_and(
        bp_ref[...] == lax.broadcasted_iota(jnp.int32, (1, N_GRAPHS), 1),
        grow < N_NODES).astype(jnp.float32)
    gm = jnp.dot(oh, gam, preferred_element_type=jnp.float32, precision=hp)
    bt = jnp.dot(oh, bet, preferred_element_type=jnp.float32, precision=hp)
    y = jnp.where(grow < N_NODES, y * gm + bt, y)
  z_ref[...] = y
  ym = jnp.where(grow < NTOT, y, 0.0)
  st_ref[0, 0:1, :] = jnp.sum(ym, axis=0, keepdims=True)
  st_ref[0, 1:2, :] = jnp.sum(ym * ym, axis=0, keepdims=True)


def _bn_body(last, z_ref, st_ref, bnw_ref, bnb_ref, out_ref):
  st = st_ref[...]
  m = jnp.sum(st[:, 0, :], axis=0, keepdims=True) * (1.0 / NTOT)
  sq = jnp.sum(st[:, 1, :], axis=0, keepdims=True) * (1.0 / NTOT)
  v = jnp.maximum(sq - m * m, 0.0)
  y = (z_ref[...] - m) * (bnw_ref[...] * lax.rsqrt(v + 1e-5)) + bnb_ref[...]
  if not last:
    y = jnp.maximum(y, 0.0)
  out_ref[...] = y


_A16 = np.zeros((16, 6), np.float32)
_B16 = np.zeros((16, 8), np.float32)
for _c in range(9):
  _A16[_c, _c // 3] = 1.0
  _B16[_c, _c % 3] = 1.0
_A16[9, 5] = 1.0
_B16[9, 0] = 1.0


def kernel(x, edge_index, edge_attr, batch, task_embs, teb, params):
  p = params
  i32 = jnp.int32
  arangeN = jnp.arange(N_NODES, dtype=i32)
  vsrc = N_NODES + batch
  pad = E_PAD - E_EXT
  pad_dst = NTOT + (jnp.arange(pad, dtype=i32) % (ACC_ROWS - NTOT))
  src_all = jnp.concatenate(
      [edge_index[0], vsrc, arangeN,
       jnp.zeros((pad,), i32)]).reshape(NW, NBLK, IDXBLK, KC)
  dst_all = jnp.concatenate(
      [edge_index[1], arangeN, vsrc,
       pad_dst]).reshape(NW, NBLK, IDXBLK, KC)
  combo = 3 * edge_attr[:, 0] + edge_attr[:, 1]
  spread = jnp.concatenate(
      [jnp.arange(E_ORIG, dtype=i32), jnp.arange(2 * N_NODES, dtype=i32),
       jnp.arange(pad, dtype=i32)]) % 256
  combo_all = (jnp.concatenate(
      [combo, jnp.full((2 * N_NODES,), 9, i32),
       jnp.zeros((pad,), i32)]) * 256 + spread).reshape(NW, NBLK, IDXBLK, KC)

  aggr_call = _sc_aggr_kernel(DIM)
  spread_tab = jnp.repeat(jnp.eye(16, DIM, dtype=jnp.float32), 256, axis=0)
  counts = aggr_call(spread_tab, combo_all, dst_all)
  cc = counts[0, :, :16] + counts[1, :, :16]

  emb2p = jnp.concatenate(
      [p['emb2'], jnp.zeros((5, DIM), jnp.float32)], axis=0)
  xcur = pl.pallas_call(
      _embed_body,
      out_shape=jax.ShapeDtypeStruct((PAD_ROWS, DIM), jnp.float32),
  )(x, p['emb1'], emb2p, teb)

  a16 = jnp.asarray(_A16)
  b16 = jnp.asarray(_B16)
  e2p = jnp.concatenate(
      [p['edge_emb2'], jnp.zeros((4, 5, DIM), jnp.float32)], axis=1)
  batch_pad = jnp.concatenate(
      [batch, jnp.zeros((PAD_ROWS - N_NODES,), i32)]).reshape(PAD_ROWS, 1)

  f32 = jnp.float32
  row_spec = pl.BlockSpec((BR, DIM), lambda i: (i, 0))
  full2 = lambda r, c: pl.BlockSpec((r, c), lambda i: (0, 0))
  for l in range(4):
    parts = aggr_call(xcur, src_all, dst_all)
    film = l in (1, 3)
    fl = l // 3
    z, st = pl.pallas_call(
        functools.partial(_mlp_body, film),
        grid=(NB,),
        in_specs=[
            pl.BlockSpec((1, BR, DIM), lambda i: (0, i, 0)),
            pl.BlockSpec((1, BR, DIM), lambda i: (1, i, 0)),
            row_spec,
            pl.BlockSpec((BR, 16), lambda i: (i, 0)),
            full2(16, 6), full2(16, 8), full2(6, DIM), full2(8, DIM),
            full2(DIM, 2 * DIM), full2(1, 2 * DIM),
            full2(2 * DIM, DIM), full2(1, DIM),
            full2(N_GRAPHS, DIM), full2(DIM, DIM), full2(1, DIM),
            full2(DIM, DIM), full2(1, DIM),
            pl.BlockSpec((BR, 1), lambda i: (i, 0)),
        ],
        out_specs=[row_spec,
                   pl.BlockSpec((1, 2, DIM), lambda i: (i, 0, 0))],
        out_shape=[jax.ShapeDtypeStruct((PAD_ROWS, DIM), f32),
                   jax.ShapeDtypeStruct((NB, 2, DIM), f32)],
    )(parts, parts, xcur, cc, a16, b16,
      p['edge_emb1'][l], e2p[l], p['W1'][l], p['b1'][l].reshape(1, -1),
      p['W2'][l], p['b2'][l].reshape(1, -1),
      task_embs, p['film_Wg'][fl], p['film_bg'][fl].reshape(1, -1),
      p['film_Wb'][fl], p['film_bb'][fl].reshape(1, -1), batch_pad)
    xcur = pl.pallas_call(
        functools.partial(_bn_body, l == 3),
        grid=(NB,),
        in_specs=[
            row_spec,
            pl.BlockSpec((NB, 2, DIM), lambda i: (0, 0, 0)),
            full2(1, DIM), full2(1, DIM),
        ],
        out_specs=row_spec,
        out_shape=jax.ShapeDtypeStruct((PAD_ROWS, DIM), f32),
    )(z, st, p['bn_w'][l].reshape(1, -1), p['bn_b'][l].reshape(1, -1))
  return xcur[:N_NODES]


# speedup vs baseline: 1.8108x; 1.8108x over previous
import functools

import jax
import jax.numpy as jnp
import numpy as np
from jax import lax
from jax.experimental import pallas as pl
from jax.experimental.pallas import tpu as pltpu
from jax.experimental.pallas import tpu_sc as plsc

N_NODES = 10000
N_GRAPHS = 256
DIM = 128
NTOT = N_NODES + N_GRAPHS
E_ORIG = 320000
E_EXT = E_ORIG + 2 * N_NODES

NC, NS = 2, 16
NW = NC * NS
KC = 128
CHUNKS = -(-E_EXT // (NW * KC))
IDXBLK = 12
NBLK = CHUNKS // IDXBLK
E_PAD = NW * KC * CHUNKS
ZROWS = 16
ZCOPIES = 41
TILE_ACC_ROWS = ZROWS * ZCOPIES
ACC_ROWS = TILE_ACC_ROWS * NS
DUMP_ROWS = TILE_ACC_ROWS
PAD_ROWS = ACC_ROWS
NB = 8
BR = PAD_ROWS // NB


def _sc_aggr_kernel(width):
  mesh = plsc.VectorSubcoreMesh(core_axis_name="c", subcore_axis_name="s")

  def body(table_hbm, src_hbm, dst_hbm, out_hbm,
           src_v, dst_v, rows_a, rows_b, acc_sh, zbuf,
           gsem_a, gsem_b, ssem_a, ssem_b, isem, zsem):
    c = lax.axis_index("c")
    s = lax.axis_index("s")
    w = c * NS + s
    zv = jnp.zeros((16,), jnp.float32)
    for i in range(ZROWS):
      for t in range(width // 16):
        zbuf[i, pl.ds(16 * t, 16)] = zv
    zdescs = []
    for r in range(ZCOPIES):
      zdescs.append(pltpu.async_copy(
          zbuf, acc_sh.at[pl.ds(s * TILE_ACC_ROWS + r * ZROWS, ZROWS)],
          zsem))
    pltpu.sync_copy(src_hbm.at[w].at[0], src_v.at[0])
    pltpu.sync_copy(dst_hbm.at[w].at[0], dst_v.at[0])
    idescs = {}

    def src_row(j):
      return src_v.at[(j // IDXBLK) % 2].at[j % IDXBLK]

    def dst_row(j):
      return dst_v.at[(j // IDXBLK) % 2].at[j % IDXBLK]

    for d in zdescs:
      d.wait()
    plsc.subcore_barrier()
    bufs = (rows_a, rows_b)
    gsems = (gsem_a, gsem_b)
    ssems = (ssem_a, ssem_b)
    gdesc = [None, None]
    sdesc = [None, None]
    gdesc[0] = pltpu.async_copy(table_hbm.at[src_row(0)], bufs[0], gsems[0])
    for j in range(CHUNKS):
      i = j % 2
      b, k = divmod(j, IDXBLK)
      if k == 2 and b + 1 < NBLK:
        nb = b + 1
        idescs[nb] = (
            pltpu.async_copy(src_hbm.at[w].at[nb], src_v.at[nb % 2], isem),
            pltpu.async_copy(dst_hbm.at[w].at[nb], dst_v.at[nb % 2], isem))
      gdesc[i].wait()
      sdesc[i] = pltpu.async_copy(bufs[i], acc_sh.at[dst_row(j)],
                                  ssems[i], add=True)
      if j + 1 < CHUNKS:
        if sdesc[1 - i] is not None:
          sdesc[1 - i].wait()
        if (j + 1) % IDXBLK == 0:
          for d in idescs.pop((j + 1) // IDXBLK):
            d.wait()
        gdesc[1 - i] = pltpu.async_copy(table_hbm.at[src_row(j + 1)],
                                        bufs[1 - i], gsems[1 - i])
    sdesc[(CHUNKS - 1) % 2].wait()
    if sdesc[CHUNKS % 2] is not None:
      sdesc[CHUNKS % 2].wait()
    plsc.subcore_barrier()
    pltpu.sync_copy(acc_sh.at[pl.ds(s * DUMP_ROWS, DUMP_ROWS)],
                    out_hbm.at[c].at[pl.ds(s * DUMP_ROWS, DUMP_ROWS)])

  return pl.kernel(
      body,
      out_type=jax.ShapeDtypeStruct((NC, ACC_ROWS, width), jnp.float32),
      mesh=mesh,
      scratch_types=[
          pltpu.VMEM((2, IDXBLK, KC), jnp.int32),
          pltpu.VMEM((2, IDXBLK, KC), jnp.int32),
          pltpu.VMEM((KC, width), jnp.float32),
          pltpu.VMEM((KC, width), jnp.float32),
          pltpu.VMEM_SHARED((ACC_ROWS, width), jnp.float32),
          pltpu.VMEM((ZROWS, width), jnp.float32),
          pltpu.SemaphoreType.DMA,
          pltpu.SemaphoreType.DMA,
          pltpu.SemaphoreType.DMA,
          pltpu.SemaphoreType.DMA,
          pltpu.SemaphoreType.DMA,
          pltpu.SemaphoreType.DMA,
      ],
  )


def _embed_body(xi_ref, emb1_ref, emb2_ref, teb_ref, out_ref):
  xi0 = xi_ref[:, 0:1]
  xi1 = xi_ref[:, 1:2]
  oh0 = (xi0 == lax.broadcasted_iota(jnp.int32, (1, 120), 1)).astype(jnp.float32)
  oh1 = (xi1 == lax.broadcasted_iota(jnp.int32, (1, 8), 1)).astype(jnp.float32)
  x0 = jnp.dot(oh0, emb1_ref[...], preferred_element_type=jnp.float32, precision=lax.Precision.HIGHEST)
  x0 = x0 + jnp.dot(oh1, emb2_ref[...], preferred_element_type=jnp.float32, precision=lax.Precision.HIGHEST)
  out_ref[pl.ds(0, N_NODES), :] = x0
  out_ref[pl.ds(N_NODES, N_GRAPHS), :] = teb_ref[...]
  out_ref[pl.ds(NTOT, PAD_ROWS - NTOT), :] = jnp.zeros(
      (PAD_ROWS - NTOT, DIM), jnp.float32)


def _mlp_body(film, p0_ref, p1_ref, xc_ref, cc_ref,
              a16_ref, b16_ref, e1_ref, e2_ref, w1_ref, b1_ref,
              w2_ref, b2_ref, te_ref, wg_ref, bg_ref,
              wb_ref, bb_ref, bp_ref, z_ref, st_ref):
  i = pl.program_id(0)
  hp = lax.Precision.HIGHEST
  e1 = e1_ref[...]
  e2 = e2_ref[...]
  t = (jnp.dot(a16_ref[...], e1, preferred_element_type=jnp.float32, precision=hp) +
       jnp.dot(b16_ref[...], e2, preferred_element_type=jnp.float32, precision=hp))
  const = e1[4:5, :] + e2[0:1, :]
  aggr = (p0_ref[0] + p1_ref[0] + xc_ref[...] + const +
          jnp.dot(cc_ref[...], t, preferred_element_type=jnp.float32, precision=hp))
  h = jnp.maximum(
      jnp.dot(aggr, w1_ref[...], preferred_element_type=jnp.float32, precision=hp) +
      b1_ref[...], 0.0)
  y = jnp.dot(h, w2_ref[...], preferred_element_type=jnp.float32, precision=hp) + b2_ref[...]
  grow = i * BR + lax.broadcasted_iota(jnp.int32, (BR, 1), 0)
  if film:
    gam = jnp.dot(te_ref[...], wg_ref[...],
                  preferred_element_type=jnp.float32, precision=hp) + bg_ref[...]
    bet = jnp.dot(te_ref[...], wb_ref[...],
                  preferred_element_type=jnp.float32, precision=hp) + bb_ref[...]
    oh = jnp.logical_and(
        bp_ref[...] == lax.broadcasted_iota(jnp.int32, (1, N_GRAPHS), 1),
        grow < N_NODES).astype(jnp.float32)
    gm = jnp.dot(oh, gam, preferred_element_type=jnp.float32, precision=hp)
    bt = jnp.dot(oh, bet, preferred_element_type=jnp.float32, precision=hp)
    y = jnp.where(grow < N_NODES, y * gm + bt, y)
  z_ref[...] = y
  ym = jnp.where(grow < NTOT, y, 0.0)
  st_ref[0, 0:1, :] = jnp.sum(ym, axis=0, keepdims=True)
  st_ref[0, 1:2, :] = jnp.sum(ym * ym, axis=0, keepdims=True)


def _bn_body(last, z_ref, st_ref, bnw_ref, bnb_ref, out_ref):
  st = st_ref[...]
  m = jnp.sum(st[:, 0, :], axis=0, keepdims=True) * (1.0 / NTOT)
  sq = jnp.sum(st[:, 1, :], axis=0, keepdims=True) * (1.0 / NTOT)
  v = jnp.maximum(sq - m * m, 0.0)
  y = (z_ref[...] - m) * (bnw_ref[...] * lax.rsqrt(v + 1e-5)) + bnb_ref[...]
  if not last:
    y = jnp.maximum(y, 0.0)
  out_ref[...] = y


_A16 = np.zeros((16, 6), np.float32)
_B16 = np.zeros((16, 8), np.float32)
for _c in range(9):
  _A16[_c, _c // 3] = 1.0
  _B16[_c, _c % 3] = 1.0
_A16[9, 5] = 1.0
_B16[9, 0] = 1.0


def kernel(x, edge_index, edge_attr, batch, task_embs, teb, params):
  p = params
  i32 = jnp.int32
  arangeN = jnp.arange(N_NODES, dtype=i32)
  vsrc = N_NODES + batch
  pad = E_PAD - E_EXT
  pad_dst = NTOT + (jnp.arange(pad, dtype=i32) % (ACC_ROWS - NTOT))
  pad_src = jnp.arange(pad, dtype=i32) % NTOT

  def _interleave(e):
    return e.reshape(CHUNKS * KC, NW).T.reshape(NW, NBLK, IDXBLK, KC)

  src_all = _interleave(jnp.concatenate(
      [edge_index[0], vsrc, arangeN, pad_src]))
  dst_all = _interleave(jnp.concatenate(
      [edge_index[1], arangeN, vsrc, pad_dst]))
  combo = 3 * edge_attr[:, 0] + edge_attr[:, 1]
  spread = jnp.concatenate(
      [jnp.arange(E_ORIG, dtype=i32), jnp.arange(2 * N_NODES, dtype=i32),
       jnp.arange(pad, dtype=i32)]) % 256
  combo_all = _interleave(jnp.concatenate(
      [combo, jnp.full((2 * N_NODES,), 9, i32),
       jnp.zeros((pad,), i32)]) * 256 + spread)

  aggr_call = _sc_aggr_kernel(DIM)
  spread_tab = jnp.repeat(jnp.eye(16, DIM, dtype=jnp.float32), 256, axis=0)
  counts = aggr_call(spread_tab, combo_all, dst_all)
  cc = counts[0, :, :16] + counts[1, :, :16]

  emb2p = jnp.concatenate(
      [p['emb2'], jnp.zeros((5, DIM), jnp.float32)], axis=0)
  xcur = pl.pallas_call(
      _embed_body,
      out_shape=jax.ShapeDtypeStruct((PAD_ROWS, DIM), jnp.float32),
  )(x, p['emb1'], emb2p, teb)

  a16 = jnp.asarray(_A16)
  b16 = jnp.asarray(_B16)
  e2p = jnp.concatenate(
      [p['edge_emb2'], jnp.zeros((4, 5, DIM), jnp.float32)], axis=1)
  batch_pad = jnp.concatenate(
      [batch, jnp.zeros((PAD_ROWS - N_NODES,), i32)]).reshape(PAD_ROWS, 1)

  f32 = jnp.float32
  row_spec = pl.BlockSpec((BR, DIM), lambda i: (i, 0))
  full2 = lambda r, c: pl.BlockSpec((r, c), lambda i: (0, 0))
  for l in range(4):
    parts = aggr_call(xcur, src_all, dst_all)
    film = l in (1, 3)
    fl = l // 3
    z, st = pl.pallas_call(
        functools.partial(_mlp_body, film),
        grid=(NB,),
        in_specs=[
            pl.BlockSpec((1, BR, DIM), lambda i: (0, i, 0)),
            pl.BlockSpec((1, BR, DIM), lambda i: (1, i, 0)),
            row_spec,
            pl.BlockSpec((BR, 16), lambda i: (i, 0)),
            full2(16, 6), full2(16, 8), full2(6, DIM), full2(8, DIM),
            full2(DIM, 2 * DIM), full2(1, 2 * DIM),
            full2(2 * DIM, DIM), full2(1, DIM),
            full2(N_GRAPHS, DIM), full2(DIM, DIM), full2(1, DIM),
            full2(DIM, DIM), full2(1, DIM),
            pl.BlockSpec((BR, 1), lambda i: (i, 0)),
        ],
        out_specs=[row_spec,
                   pl.BlockSpec((1, 2, DIM), lambda i: (i, 0, 0))],
        out_shape=[jax.ShapeDtypeStruct((PAD_ROWS, DIM), f32),
                   jax.ShapeDtypeStruct((NB, 2, DIM), f32)],
    )(parts, parts, xcur, cc, a16, b16,
      p['edge_emb1'][l], e2p[l], p['W1'][l], p['b1'][l].reshape(1, -1),
      p['W2'][l], p['b2'][l].reshape(1, -1),
      task_embs, p['film_Wg'][fl], p['film_bg'][fl].reshape(1, -1),
      p['film_Wb'][fl], p['film_bb'][fl].reshape(1, -1), batch_pad)
    xcur = pl.pallas_call(
        functools.partial(_bn_body, l == 3),
        grid=(NB,),
        in_specs=[
            row_spec,
            pl.BlockSpec((NB, 2, DIM), lambda i: (0, 0, 0)),
            full2(1, DIM), full2(1, DIM),
        ],
        out_specs=row_spec,
        out_shape=jax.ShapeDtypeStruct((PAD_ROWS, DIM), f32),
    )(z, st, p['bn_w'][l].reshape(1, -1), p['bn_b'][l].reshape(1, -1))
  return xcur[:N_NODES]


# speedup vs baseline: 1.9865x; 1.0971x over previous
import functools

import jax
import jax.numpy as jnp
import numpy as np
from jax import lax
from jax.experimental import pallas as pl
from jax.experimental.pallas import tpu as pltpu
from jax.experimental.pallas import tpu_sc as plsc

N_NODES = 10000
N_GRAPHS = 256
DIM = 128
NTOT = N_NODES + N_GRAPHS
E_ORIG = 320000
E_EXT = E_ORIG + 2 * N_NODES

NC, NS = 2, 16
NW = NC * NS
KC = 128
CHUNKS = -(-E_EXT // (NW * KC))
IDXBLK = 12
NBLK = CHUNKS // IDXBLK
E_PAD = NW * KC * CHUNKS
ZROWS = 16
ZCOPIES = 41
TILE_ACC_ROWS = ZROWS * ZCOPIES
ACC_ROWS = TILE_ACC_ROWS * NS
DUMP_ROWS = TILE_ACC_ROWS
PAD_ROWS = ACC_ROWS
NB = 8
BR = PAD_ROWS // NB


def _sc_aggr_kernel(width):
  mesh = plsc.VectorSubcoreMesh(core_axis_name="c", subcore_axis_name="s")

  def body(table_hbm, src_hbm, dst_hbm, out_hbm,
           src_v, dst_v, rows_a, rows_b, acc_sh, zbuf,
           gsem_a, gsem_b, ssem_a, ssem_b, isem, zsem):
    c = lax.axis_index("c")
    s = lax.axis_index("s")
    w = c * NS + s
    zv = jnp.zeros((16,), jnp.float32)
    for i in range(ZROWS):
      for t in range(width // 16):
        zbuf[i, pl.ds(16 * t, 16)] = zv
    zdescs = []
    for r in range(ZCOPIES):
      zdescs.append(pltpu.async_copy(
          zbuf, acc_sh.at[pl.ds(s * TILE_ACC_ROWS + r * ZROWS, ZROWS)],
          zsem))
    pltpu.sync_copy(src_hbm.at[w].at[0], src_v.at[0])
    pltpu.sync_copy(dst_hbm.at[w].at[0], dst_v.at[0])
    idescs = {}

    def src_row(j):
      return src_v.at[(j // IDXBLK) % 2].at[j % IDXBLK]

    def dst_row(j):
      return dst_v.at[(j // IDXBLK) % 2].at[j % IDXBLK]

    for d in zdescs:
      d.wait()
    plsc.subcore_barrier()
    bufs = (rows_a, rows_b)
    gsems = (gsem_a, gsem_b)
    ssems = (ssem_a, ssem_b)
    gdesc = [None, None]
    sdesc = [None, None]
    gdesc[0] = pltpu.async_copy(table_hbm.at[src_row(0)], bufs[0], gsems[0])
    for j in range(CHUNKS):
      i = j % 2
      b, k = divmod(j, IDXBLK)
      if k == 2 and b + 1 < NBLK:
        nb = b + 1
        idescs[nb] = (
            pltpu.async_copy(src_hbm.at[w].at[nb], src_v.at[nb % 2], isem),
            pltpu.async_copy(dst_hbm.at[w].at[nb], dst_v.at[nb % 2], isem))
      gdesc[i].wait()
      sdesc[i] = pltpu.async_copy(bufs[i], acc_sh.at[dst_row(j)],
                                  ssems[i], add=True)
      if j + 1 < CHUNKS:
        if sdesc[1 - i] is not None:
          sdesc[1 - i].wait()
        if (j + 1) % IDXBLK == 0:
          for d in idescs.pop((j + 1) // IDXBLK):
            d.wait()
        gdesc[1 - i] = pltpu.async_copy(table_hbm.at[src_row(j + 1)],
                                        bufs[1 - i], gsems[1 - i])
    sdesc[(CHUNKS - 1) % 2].wait()
    if sdesc[CHUNKS % 2] is not None:
      sdesc[CHUNKS % 2].wait()
    plsc.subcore_barrier()
    pltpu.sync_copy(acc_sh.at[pl.ds(s * DUMP_ROWS, DUMP_ROWS)],
                    out_hbm.at[c].at[pl.ds(s * DUMP_ROWS, DUMP_ROWS)])

  return pl.kernel(
      body,
      out_type=jax.ShapeDtypeStruct((NC, ACC_ROWS, width), jnp.float32),
      mesh=mesh,
      scratch_types=[
          pltpu.VMEM((2, IDXBLK, KC), jnp.int32),
          pltpu.VMEM((2, IDXBLK, KC), jnp.int32),
          pltpu.VMEM((KC, width), jnp.float32),
          pltpu.VMEM((KC, width), jnp.float32),
          pltpu.VMEM_SHARED((ACC_ROWS, width), jnp.float32),
          pltpu.VMEM((ZROWS, width), jnp.float32),
          pltpu.SemaphoreType.DMA,
          pltpu.SemaphoreType.DMA,
          pltpu.SemaphoreType.DMA,
          pltpu.SemaphoreType.DMA,
          pltpu.SemaphoreType.DMA,
          pltpu.SemaphoreType.DMA,
      ],
  )


def _embed_body(xi_ref, emb1_ref, emb2_ref, teb_ref, out_ref):
  xi0 = xi_ref[:, 0:1]
  xi1 = xi_ref[:, 1:2]
  oh0 = (xi0 == lax.broadcasted_iota(jnp.int32, (1, 120), 1)).astype(jnp.float32)
  oh1 = (xi1 == lax.broadcasted_iota(jnp.int32, (1, 8), 1)).astype(jnp.float32)
  x0 = jnp.dot(oh0, emb1_ref[...], preferred_element_type=jnp.float32, precision=lax.Precision.HIGHEST)
  x0 = x0 + jnp.dot(oh1, emb2_ref[...], preferred_element_type=jnp.float32, precision=lax.Precision.HIGHEST)
  out_ref[pl.ds(0, N_NODES), :] = x0
  out_ref[pl.ds(N_NODES, N_GRAPHS), :] = teb_ref[...]
  out_ref[pl.ds(NTOT, PAD_ROWS - NTOT), :] = jnp.zeros(
      (PAD_ROWS - NTOT, DIM), jnp.float32)


def _mlp_body(film, p0_ref, p1_ref, xc_ref, cc_ref,
              a16_ref, b16_ref, e1_ref, e2_ref, w1_ref, b1_ref,
              w2_ref, b2_ref, te_ref, wg_ref, bg_ref,
              wb_ref, bb_ref, bp_ref, z_ref, st_ref):
  i = pl.program_id(0)
  hp = lax.Precision.HIGHEST
  e1 = e1_ref[...]
  e2 = e2_ref[...]
  t = (jnp.dot(a16_ref[...], e1, preferred_element_type=jnp.float32, precision=hp) +
       jnp.dot(b16_ref[...], e2, preferred_element_type=jnp.float32, precision=hp))
  const = e1[4:5, :] + e2[0:1, :]
  aggr = (p0_ref[0] + p1_ref[0] + xc_ref[...] + const +
          jnp.dot(cc_ref[...], t, preferred_element_type=jnp.float32, precision=hp))
  h = jnp.maximum(
      jnp.dot(aggr, w1_ref[...], preferred_element_type=jnp.float32) +
      b1_ref[...], 0.0)
  y = jnp.dot(h, w2_ref[...], preferred_element_type=jnp.float32) + b2_ref[...]
  grow = i * BR + lax.broadcasted_iota(jnp.int32, (BR, 1), 0)
  if film:
    gam = jnp.dot(te_ref[...], wg_ref[...],
                  preferred_element_type=jnp.float32) + bg_ref[...]
    bet = jnp.dot(te_ref[...], wb_ref[...],
                  preferred_element_type=jnp.float32) + bb_ref[...]
    oh = jnp.logical_and(
        bp_ref[...] == lax.broadcasted_iota(jnp.int32, (1, N_GRAPHS), 1),
        grow < N_NODES).astype(jnp.float32)
    gm = jnp.dot(oh, gam, preferred_element_type=jnp.float32, precision=hp)
    bt = jnp.dot(oh, bet, preferred_element_type=jnp.float32, precision=hp)
    y = jnp.where(grow < N_NODES, y * gm + bt, y)
  z_ref[...] = y
  ym = jnp.where(grow < NTOT, y, 0.0)
  st_ref[0, 0:1, :] = jnp.sum(ym, axis=0, keepdims=True)
  st_ref[0, 1:2, :] = jnp.sum(ym * ym, axis=0, keepdims=True)


def _bn_body(last, z_ref, st_ref, bnw_ref, bnb_ref, out_ref):
  st = st_ref[...]
  m = jnp.sum(st[:, 0, :], axis=0, keepdims=True) * (1.0 / NTOT)
  sq = jnp.sum(st[:, 1, :], axis=0, keepdims=True) * (1.0 / NTOT)
  v = jnp.maximum(sq - m * m, 0.0)
  y = (z_ref[...] - m) * (bnw_ref[...] * lax.rsqrt(v + 1e-5)) + bnb_ref[...]
  if not last:
    y = jnp.maximum(y, 0.0)
  out_ref[...] = y


_A16 = np.zeros((16, 6), np.float32)
_B16 = np.zeros((16, 8), np.float32)
for _c in range(9):
  _A16[_c, _c // 3] = 1.0
  _B16[_c, _c % 3] = 1.0
_A16[9, 5] = 1.0
_B16[9, 0] = 1.0


def kernel(x, edge_index, edge_attr, batch, task_embs, teb, params):
  p = params
  i32 = jnp.int32
  arangeN = jnp.arange(N_NODES, dtype=i32)
  vsrc = N_NODES + batch
  pad = E_PAD - E_EXT
  pad_dst = NTOT + (jnp.arange(pad, dtype=i32) % (ACC_ROWS - NTOT))
  pad_src = jnp.arange(pad, dtype=i32) % NTOT

  def _interleave(e):
    return e.reshape(CHUNKS * KC, NW).T.reshape(NW, NBLK, IDXBLK, KC)

  src_all = _interleave(jnp.concatenate(
      [edge_index[0], vsrc, arangeN, pad_src]))
  dst_all = _interleave(jnp.concatenate(
      [edge_index[1], arangeN, vsrc, pad_dst]))
  combo = 3 * edge_attr[:, 0] + edge_attr[:, 1]
  spread = jnp.concatenate(
      [jnp.arange(E_ORIG, dtype=i32), jnp.arange(2 * N_NODES, dtype=i32),
       jnp.arange(pad, dtype=i32)]) % 256
  combo_all = _interleave(jnp.concatenate(
      [combo, jnp.full((2 * N_NODES,), 9, i32),
       jnp.zeros((pad,), i32)]) * 256 + spread)

  aggr_call = _sc_aggr_kernel(DIM)
  spread_tab = jnp.repeat(jnp.eye(16, DIM, dtype=jnp.float32), 256, axis=0)
  counts = aggr_call(spread_tab, combo_all, dst_all)
  cc = counts[0, :, :16] + counts[1, :, :16]

  emb2p = jnp.concatenate(
      [p['emb2'], jnp.zeros((5, DIM), jnp.float32)], axis=0)
  xcur = pl.pallas_call(
      _embed_body,
      out_shape=jax.ShapeDtypeStruct((PAD_ROWS, DIM), jnp.float32),
  )(x, p['emb1'], emb2p, teb)

  a16 = jnp.asarray(_A16)
  b16 = jnp.asarray(_B16)
  e2p = jnp.concatenate(
      [p['edge_emb2'], jnp.zeros((4, 5, DIM), jnp.float32)], axis=1)
  batch_pad = jnp.concatenate(
      [batch, jnp.zeros((PAD_ROWS - N_NODES,), i32)]).reshape(PAD_ROWS, 1)

  f32 = jnp.float32
  row_spec = pl.BlockSpec((BR, DIM), lambda i: (i, 0))
  full2 = lambda r, c: pl.BlockSpec((r, c), lambda i: (0, 0))
  for l in range(4):
    parts = aggr_call(xcur, src_all, dst_all)
    film = l in (1, 3)
    fl = l // 3
    z, st = pl.pallas_call(
        functools.partial(_mlp_body, film),
        grid=(NB,),
        in_specs=[
            pl.BlockSpec((1, BR, DIM), lambda i: (0, i, 0)),
            pl.BlockSpec((1, BR, DIM), lambda i: (1, i, 0)),
            row_spec,
            pl.BlockSpec((BR, 16), lambda i: (i, 0)),
            full2(16, 6), full2(16, 8), full2(6, DIM), full2(8, DIM),
            full2(DIM, 2 * DIM), full2(1, 2 * DIM),
            full2(2 * DIM, DIM), full2(1, DIM),
            full2(N_GRAPHS, DIM), full2(DIM, DIM), full2(1, DIM),
            full2(DIM, DIM), full2(1, DIM),
            pl.BlockSpec((BR, 1), lambda i: (i, 0)),
        ],
        out_specs=[row_spec,
                   pl.BlockSpec((1, 2, DIM), lambda i: (i, 0, 0))],
        out_shape=[jax.ShapeDtypeStruct((PAD_ROWS, DIM), f32),
                   jax.ShapeDtypeStruct((NB, 2, DIM), f32)],
    )(parts, parts, xcur, cc, a16, b16,
      p['edge_emb1'][l], e2p[l], p['W1'][l], p['b1'][l].reshape(1, -1),
      p['W2'][l], p['b2'][l].reshape(1, -1),
      task_embs, p['film_Wg'][fl], p['film_bg'][fl].reshape(1, -1),
      p['film_Wb'][fl], p['film_bb'][fl].reshape(1, -1), batch_pad)
    xcur = pl.pallas_call(
        functools.partial(_bn_body, l == 3),
        grid=(NB,),
        in_specs=[
            row_spec,
            pl.BlockSpec((NB, 2, DIM), lambda i: (0, 0, 0)),
            full2(1, DIM), full2(1, DIM),
        ],
        out_specs=row_spec,
        out_shape=jax.ShapeDtypeStruct((PAD_ROWS, DIM), f32),
    )(z, st, p['bn_w'][l].reshape(1, -1), p['bn_b'][l].reshape(1, -1))
  return xcur[:N_NODES]


# speedup vs baseline: 2.0325x; 1.0231x over previous
import functools

import jax
import jax.numpy as jnp
import numpy as np
from jax import lax
from jax.experimental import pallas as pl
from jax.experimental.pallas import tpu as pltpu
from jax.experimental.pallas import tpu_sc as plsc

N_NODES = 10000
N_GRAPHS = 256
DIM = 128
NTOT = N_NODES + N_GRAPHS
E_ORIG = 320000
E_EXT = E_ORIG + 2 * N_NODES

NC, NS = 2, 16
NW = NC * NS
KC = 128
CHUNKS = -(-E_EXT // (NW * KC))
IDXBLK = 12
NBLK = CHUNKS // IDXBLK
E_PAD = NW * KC * CHUNKS
ZROWS = 16
ZCOPIES = 41
TILE_ACC_ROWS = ZROWS * ZCOPIES
ACC_ROWS = TILE_ACC_ROWS * NS
DUMP_ROWS = TILE_ACC_ROWS
PAD_ROWS = ACC_ROWS
NB = 8
BR = PAD_ROWS // NB


def _sc_aggr_kernel(width):
  mesh = plsc.VectorSubcoreMesh(core_axis_name="c", subcore_axis_name="s")

  def body(table_hbm, src_hbm, dst_hbm, out_hbm,
           src_v, dst_v, rows_a, rows_b, acc_sh, zbuf,
           gsem_a, gsem_b, ssem_a, ssem_b, isem, zsem):
    c = lax.axis_index("c")
    s = lax.axis_index("s")
    w = c * NS + s
    zv = jnp.zeros((16,), jnp.float32)
    for i in range(ZROWS):
      for t in range(width // 16):
        zbuf[i, pl.ds(16 * t, 16)] = zv
    zdescs = []
    for r in range(ZCOPIES):
      zdescs.append(pltpu.async_copy(
          zbuf, acc_sh.at[pl.ds(s * TILE_ACC_ROWS + r * ZROWS, ZROWS)],
          zsem))
    pltpu.sync_copy(src_hbm.at[w].at[0], src_v.at[0])
    pltpu.sync_copy(dst_hbm.at[w].at[0], dst_v.at[0])
    idescs = {}

    def src_row(j):
      return src_v.at[(j // IDXBLK) % 2].at[j % IDXBLK]

    def dst_row(j):
      return dst_v.at[(j // IDXBLK) % 2].at[j % IDXBLK]

    for d in zdescs:
      d.wait()
    plsc.subcore_barrier()
    bufs = (rows_a, rows_b)
    gsems = (gsem_a, gsem_b)
    ssems = (ssem_a, ssem_b)
    gdesc = [None, None]
    sdesc = [None, None]
    gdesc[0] = pltpu.async_copy(table_hbm.at[src_row(0)], bufs[0], gsems[0])
    for j in range(CHUNKS):
      i = j % 2
      b, k = divmod(j, IDXBLK)
      if k == 2 and b + 1 < NBLK:
        nb = b + 1
        idescs[nb] = (
            pltpu.async_copy(src_hbm.at[w].at[nb], src_v.at[nb % 2], isem),
            pltpu.async_copy(dst_hbm.at[w].at[nb], dst_v.at[nb % 2], isem))
      gdesc[i].wait()
      sdesc[i] = pltpu.async_copy(bufs[i], acc_sh.at[dst_row(j)],
                                  ssems[i], add=True)
      if j + 1 < CHUNKS:
        if sdesc[1 - i] is not None:
          sdesc[1 - i].wait()
        if (j + 1) % IDXBLK == 0:
          for d in idescs.pop((j + 1) // IDXBLK):
            d.wait()
        gdesc[1 - i] = pltpu.async_copy(table_hbm.at[src_row(j + 1)],
                                        bufs[1 - i], gsems[1 - i])
    sdesc[(CHUNKS - 1) % 2].wait()
    if sdesc[CHUNKS % 2] is not None:
      sdesc[CHUNKS % 2].wait()
    plsc.subcore_barrier()
    pltpu.sync_copy(acc_sh.at[pl.ds(s * DUMP_ROWS, DUMP_ROWS)],
                    out_hbm.at[c].at[pl.ds(s * DUMP_ROWS, DUMP_ROWS)])

  return pl.kernel(
      body,
      out_type=jax.ShapeDtypeStruct((NC, ACC_ROWS, width), jnp.float32),
      mesh=mesh,
      scratch_types=[
          pltpu.VMEM((2, IDXBLK, KC), jnp.int32),
          pltpu.VMEM((2, IDXBLK, KC), jnp.int32),
          pltpu.VMEM((KC, width), jnp.float32),
          pltpu.VMEM((KC, width), jnp.float32),
          pltpu.VMEM_SHARED((ACC_ROWS, width), jnp.float32),
          pltpu.VMEM((ZROWS, width), jnp.float32),
          pltpu.SemaphoreType.DMA,
          pltpu.SemaphoreType.DMA,
          pltpu.SemaphoreType.DMA,
          pltpu.SemaphoreType.DMA,
          pltpu.SemaphoreType.DMA,
          pltpu.SemaphoreType.DMA,
      ],
  )


def _embed_body(xi_ref, emb1_ref, emb2_ref, teb_ref, out_ref):
  xi0 = xi_ref[:, 0:1]
  xi1 = xi_ref[:, 1:2]
  oh0 = (xi0 == lax.broadcasted_iota(jnp.int32, (1, 120), 1)).astype(jnp.float32)
  oh1 = (xi1 == lax.broadcasted_iota(jnp.int32, (1, 8), 1)).astype(jnp.float32)
  x0 = jnp.dot(oh0, emb1_ref[...], preferred_element_type=jnp.float32, precision=lax.Precision.HIGHEST)
  x0 = x0 + jnp.dot(oh1, emb2_ref[...], preferred_element_type=jnp.float32, precision=lax.Precision.HIGHEST)
  out_ref[pl.ds(0, N_NODES), :] = x0
  out_ref[pl.ds(N_NODES, N_GRAPHS), :] = teb_ref[...]
  out_ref[pl.ds(NTOT, PAD_ROWS - NTOT), :] = jnp.zeros(
      (PAD_ROWS - NTOT, DIM), jnp.float32)


def _layer_body(film, last, p0_ref, p1_ref, xc_ref, cc_ref,
                a16_ref, b16_ref, e1_ref, e2_ref, w1_ref, b1_ref,
                w2_ref, b2_ref, bnw_ref, bnb_ref, te_ref, wg_ref, bg_ref,
                wb_ref, bb_ref, bp_ref, out_ref, zbuf, stat):
  i = pl.program_id(0)
  hp = lax.Precision.HIGHEST

  @pl.when(i < NB)
  def _compute():
    e1 = e1_ref[...]
    e2 = e2_ref[...]
    t = (jnp.dot(a16_ref[...], e1, preferred_element_type=jnp.float32, precision=hp) +
         jnp.dot(b16_ref[...], e2, preferred_element_type=jnp.float32, precision=hp))
    const = e1[4:5, :] + e2[0:1, :]
    aggr = (p0_ref[0] + p1_ref[0] + xc_ref[...] + const +
            jnp.dot(cc_ref[...], t, preferred_element_type=jnp.float32, precision=hp))
    h = jnp.maximum(
        jnp.dot(aggr, w1_ref[...], preferred_element_type=jnp.float32) +
        b1_ref[...], 0.0)
    y = jnp.dot(h, w2_ref[...], preferred_element_type=jnp.float32) + b2_ref[...]
    grow = i * BR + lax.broadcasted_iota(jnp.int32, (BR, 1), 0)
    if film:
      gam = jnp.dot(te_ref[...], wg_ref[...],
                    preferred_element_type=jnp.float32) + bg_ref[...]
      bet = jnp.dot(te_ref[...], wb_ref[...],
                    preferred_element_type=jnp.float32) + bb_ref[...]
      oh = jnp.logical_and(
          bp_ref[...] == lax.broadcasted_iota(jnp.int32, (1, N_GRAPHS), 1),
          grow < N_NODES).astype(jnp.float32)
      gm = jnp.dot(oh, gam, preferred_element_type=jnp.float32, precision=hp)
      bt = jnp.dot(oh, bet, preferred_element_type=jnp.float32, precision=hp)
      y = jnp.where(grow < N_NODES, y * gm + bt, y)
    zbuf[pl.ds(i * BR, BR), :] = y
    ym = jnp.where(grow < NTOT, y, 0.0)
    s0 = jnp.sum(ym, axis=0, keepdims=True)
    s1 = jnp.sum(ym * ym, axis=0, keepdims=True)

    @pl.when(i == 0)
    def _init():
      stat[0:1, :] = s0
      stat[1:2, :] = s1

    @pl.when(i > 0)
    def _accum():
      stat[0:1, :] = stat[0:1, :] + s0
      stat[1:2, :] = stat[1:2, :] + s1

  @pl.when(i >= NB)
  def _normalize():
    m = stat[0:1, :] * (1.0 / NTOT)
    sq = stat[1:2, :] * (1.0 / NTOT)
    v = jnp.maximum(sq - m * m, 0.0)
    y = ((zbuf[pl.ds((i - NB) * BR, BR), :] - m) *
         (bnw_ref[...] * lax.rsqrt(v + 1e-5)) + bnb_ref[...])
    if not last:
      y = jnp.maximum(y, 0.0)
    out_ref[...] = y


_A16 = np.zeros((16, 6), np.float32)
_B16 = np.zeros((16, 8), np.float32)
for _c in range(9):
  _A16[_c, _c // 3] = 1.0
  _B16[_c, _c % 3] = 1.0
_A16[9, 5] = 1.0
_B16[9, 0] = 1.0


def kernel(x, edge_index, edge_attr, batch, task_embs, teb, params):
  p = params
  i32 = jnp.int32
  arangeN = jnp.arange(N_NODES, dtype=i32)
  vsrc = N_NODES + batch
  pad = E_PAD - E_EXT
  pad_dst = NTOT + (jnp.arange(pad, dtype=i32) % (ACC_ROWS - NTOT))
  pad_src = jnp.arange(pad, dtype=i32) % NTOT

  def _interleave(e):
    return e.reshape(CHUNKS * KC, NW).T.reshape(NW, NBLK, IDXBLK, KC)

  src_all = _interleave(jnp.concatenate(
      [edge_index[0], vsrc, arangeN, pad_src]))
  dst_all = _interleave(jnp.concatenate(
      [edge_index[1], arangeN, vsrc, pad_dst]))
  combo = 3 * edge_attr[:, 0] + edge_attr[:, 1]
  spread = jnp.concatenate(
      [jnp.arange(E_ORIG, dtype=i32), jnp.arange(2 * N_NODES, dtype=i32),
       jnp.arange(pad, dtype=i32)]) % 256
  combo_all = _interleave(jnp.concatenate(
      [combo, jnp.full((2 * N_NODES,), 9, i32),
       jnp.zeros((pad,), i32)]) * 256 + spread)

  aggr_call = _sc_aggr_kernel(DIM)
  spread_tab = jnp.repeat(jnp.eye(16, DIM, dtype=jnp.float32), 256, axis=0)
  counts = aggr_call(spread_tab, combo_all, dst_all)
  cc = counts[0, :, :16] + counts[1, :, :16]

  emb2p = jnp.concatenate(
      [p['emb2'], jnp.zeros((5, DIM), jnp.float32)], axis=0)
  xcur = pl.pallas_call(
      _embed_body,
      out_shape=jax.ShapeDtypeStruct((PAD_ROWS, DIM), jnp.float32),
  )(x, p['emb1'], emb2p, teb)

  a16 = jnp.asarray(_A16)
  b16 = jnp.asarray(_B16)
  e2p = jnp.concatenate(
      [p['edge_emb2'], jnp.zeros((4, 5, DIM), jnp.float32)], axis=1)
  batch_pad = jnp.concatenate(
      [batch, jnp.zeros((PAD_ROWS - N_NODES,), i32)]).reshape(PAD_ROWS, 1)

  f32 = jnp.float32
  clamp1 = lambda i: jnp.minimum(i, NB - 1)
  blk_spec = pl.BlockSpec((BR, DIM), lambda i: (clamp1(i), 0))
  full2 = lambda r, c: pl.BlockSpec((r, c), lambda i: (0, 0))
  for l in range(4):
    parts = aggr_call(xcur, src_all, dst_all)
    film = l in (1, 3)
    fl = l // 3
    xcur = pl.pallas_call(
        functools.partial(_layer_body, film, l == 3),
        grid=(2 * NB,),
        in_specs=[
            pl.BlockSpec((1, BR, DIM), lambda i: (0, clamp1(i), 0)),
            pl.BlockSpec((1, BR, DIM), lambda i: (1, clamp1(i), 0)),
            blk_spec,
            pl.BlockSpec((BR, 16), lambda i: (clamp1(i), 0)),
            full2(16, 6), full2(16, 8), full2(6, DIM), full2(8, DIM),
            full2(DIM, 2 * DIM), full2(1, 2 * DIM),
            full2(2 * DIM, DIM), full2(1, DIM),
            full2(1, DIM), full2(1, DIM),
            full2(N_GRAPHS, DIM), full2(DIM, DIM), full2(1, DIM),
            full2(DIM, DIM), full2(1, DIM),
            pl.BlockSpec((BR, 1), lambda i: (clamp1(i), 0)),
        ],
        out_specs=pl.BlockSpec((BR, DIM),
                               lambda i: (jnp.maximum(i - NB, 0), 0)),
        out_shape=jax.ShapeDtypeStruct((PAD_ROWS, DIM), f32),
        scratch_shapes=[pltpu.VMEM((PAD_ROWS, DIM), f32),
                        pltpu.VMEM((2, DIM), f32)],
    )(parts, parts, xcur, cc, a16, b16,
      p['edge_emb1'][l], e2p[l], p['W1'][l], p['b1'][l].reshape(1, -1),
      p['W2'][l], p['b2'][l].reshape(1, -1),
      p['bn_w'][l].reshape(1, -1), p['bn_b'][l].reshape(1, -1),
      task_embs, p['film_Wg'][fl], p['film_bg'][fl].reshape(1, -1),
      p['film_Wb'][fl], p['film_bb'][fl].reshape(1, -1), batch_pad)
  return xcur[:N_NODES]
